# baseline (speedup 1.0000x reference)
import functools

import jax
import jax.numpy as jnp
import numpy as np
from jax import lax
from jax.experimental import pallas as pl
from jax.experimental.pallas import tpu as pltpu
from jax.experimental.pallas import tpu_sc as plsc

B, L, D = 16384, 200, 64
NROWS = B * L
NW = 32
LT = L // 8
BT = B // 128
BPW = BT // NW
SCALE = 1.0 / 0.75


def _keep_mask_flat(seed_hi: int, seed_lo: int, n: int, thresh_mant: int) -> np.ndarray:
    rot_a = (13, 15, 26, 6)
    rot_b = (17, 29, 16, 24)
    k0 = np.uint32(seed_hi)
    k1 = np.uint32(seed_lo)
    k2 = np.uint32(k0 ^ k1 ^ np.uint32(0x1BD11BDA))
    out = np.empty(n, dtype=bool)
    chunk = 1 << 24
    tmp = np.empty(chunk, dtype=np.uint32)
    for s in range(0, n, chunk):
        e = min(s + chunk, n)
        m = e - s
        x1 = np.arange(s, e, dtype=np.uint32)
        x0 = np.full(m, k0, dtype=np.uint32)
        x1 += k1
        t = tmp[:m]

        def rounds(rots):
            for r in rots:
                np.add(x0, x1, out=x0)
                np.left_shift(x1, np.uint32(r), out=t)
                np.right_shift(x1, np.uint32(32 - r), out=x1)
                np.bitwise_or(x1, t, out=x1)
                np.bitwise_xor(x1, x0, out=x1)

        rounds(rot_a)
        x0 += k1
        x1 += np.uint32(k2 + np.uint32(1))
        rounds(rot_b)
        x0 += k2
        x1 += np.uint32(k0 + np.uint32(2))
        rounds(rot_a)
        x0 += k0
        x1 += np.uint32(k1 + np.uint32(3))
        rounds(rot_b)
        x0 += k1
        x1 += np.uint32(k2 + np.uint32(4))
        rounds(rot_a)
        x0 += k2
        x1 += np.uint32(k0 + np.uint32(5))
        x0 ^= x1
        x0 >>= np.uint32(9)
        np.less(x0, np.uint32(thresh_mant), out=out[s:e])
    return out


def _dropout_mask_words() -> np.ndarray:
    keep = _keep_mask_flat(0, 42, B * L * D, int(0.75 * (1 << 23)))
    k4 = keep.reshape(BT, 128, L, D).transpose(0, 2, 3, 1)
    packed = np.packbits(
        k4.reshape(BT, L, D, 4, 32).astype(np.uint8), axis=-1, bitorder="little"
    )
    return (
        packed.reshape(BT * L * D * 4, 4).view(np.uint32).astype(np.int32).reshape(-1)
    )


_MASK_WORDS = _dropout_mask_words()


@functools.partial(
    pl.kernel,
    out_type=jax.ShapeDtypeStruct((L, 8, BT, 1024), jnp.float32),
    mesh=plsc.VectorSubcoreMesh(core_axis_name="c", subcore_axis_name="s"),
    compiler_params=pltpu.CompilerParams(
        use_tc_tiling_on_sc=False, needs_layout_passes=False
    ),
    scratch_types=[
        pltpu.VMEM((1024,), jnp.int32),
        pltpu.VMEM((2048,), jnp.int32),
        pltpu.VMEM((256, D), jnp.float32),
        pltpu.VMEM((8, 1024), jnp.float32),
        pltpu.SemaphoreType.DMA,
    ],
)
def _emb_dropout(xp_hbm, mw_hbm, table_hbm, out_hbm, idx_v, mw_v, rows_v, out_v, sem):
    wid = lax.axis_index("s") * 2 + lax.axis_index("c")
    iota = lax.iota(jnp.int32, 16)
    row_idx = [[iota + (li * 128 + bg * 16) for bg in range(8)] for li in range(2)]
    shl = [31 - iota, 15 - iota]
    zero = jnp.zeros((16,), jnp.float32)
    scale = jnp.full((16,), SCALE, jnp.float32)

    for j in range(BPW):
        bblk = wid * BPW + j

        def lt_body(lt, c0, bblk=bblk):
            pltpu.sync_copy(xp_hbm.at[pl.ds(((lt * BT + bblk) * 8) * 128, 1024)], idx_v)
            pltpu.sync_copy(mw_hbm.at[pl.ds((bblk * L + lt * 8) * 256, 2048)], mw_v)

            def p_body(p, c1, bblk=bblk, lt=lt):
                cp0 = pltpu.async_copy(
                    table_hbm.at[idx_v.at[pl.ds(p * 256, 128)]],
                    rows_v.at[pl.ds(0, 128)],
                    sem,
                )
                cp1 = pltpu.async_copy(
                    table_hbm.at[idx_v.at[pl.ds(p * 256 + 128, 128)]],
                    rows_v.at[pl.ds(128, 128)],
                    sem,
                )
                cp0.wait()
                cp1.wait()
                for li in range(2):
                    lloc = p * 2 + li

                    def dblk_body(db, c2, li=li, lloc=lloc):
                        ww = mw_v[pl.ds(lloc * 256 + db * 16, 16)]
                        for di in range(4):
                            d = db * 4 + di
                            col = jnp.broadcast_to(d, (16,))
                            for bg in range(8):
                                v = plsc.load_gather(rows_v, [row_idx[li][bg], col])
                                w = jnp.broadcast_to(ww[di * 4 + bg // 2], (16,))
                                keep = lax.shift_left(w, shl[bg % 2]) < 0
                                sv = lax.select(keep, v * scale, zero)
                                out_v[d // 8, pl.ds((d % 8) * 128 + bg * 16, 16)] = sv
                        return c2

                    lax.fori_loop(0, 16, dblk_body, 0)
                    l = lt * 8 + lloc
                    pltpu.sync_copy(out_v, out_hbm.at[l, slice(None), bblk])
                return c1

            lax.fori_loop(0, 4, p_body, 0)
            return c0

        lax.fori_loop(0, LT, lt_body, 0)


def kernel(x, table):
    xp = (
        x.astype(jnp.int32)
        .transpose(1, 0)
        .reshape(LT, 8, BT, 128)
        .transpose(0, 2, 1, 3)
        .reshape(-1)
    )
    out5 = _emb_dropout(xp, _MASK_WORDS, table)
    return (
        out5.reshape(L, 8, BT, 8, 128).transpose(2, 4, 0, 1, 3).reshape(B, L, D)
    )


# speedup vs baseline: 1.3494x; 1.3494x over previous
import functools

import jax
import jax.numpy as jnp
import numpy as np
from jax import lax
from jax.experimental import pallas as pl
from jax.experimental.pallas import tpu as pltpu
from jax.experimental.pallas import tpu_sc as plsc

B, L, D = 16384, 200, 64
NROWS = B * L
NW = 32
LT = L // 8
BT = B // 128
BPW = BT // NW
SCALE = 1.0 / 0.75


def _keep_mask_flat(seed_hi: int, seed_lo: int, n: int, thresh_mant: int) -> np.ndarray:
    rot_a = (13, 15, 26, 6)
    rot_b = (17, 29, 16, 24)
    k0 = np.uint32(seed_hi)
    k1 = np.uint32(seed_lo)
    k2 = np.uint32(k0 ^ k1 ^ np.uint32(0x1BD11BDA))
    out = np.empty(n, dtype=bool)
    chunk = 1 << 24
    tmp = np.empty(chunk, dtype=np.uint32)
    for s in range(0, n, chunk):
        e = min(s + chunk, n)
        m = e - s
        x1 = np.arange(s, e, dtype=np.uint32)
        x0 = np.full(m, k0, dtype=np.uint32)
        x1 += k1
        t = tmp[:m]

        def rounds(rots):
            for r in rots:
                np.add(x0, x1, out=x0)
                np.left_shift(x1, np.uint32(r), out=t)
                np.right_shift(x1, np.uint32(32 - r), out=x1)
                np.bitwise_or(x1, t, out=x1)
                np.bitwise_xor(x1, x0, out=x1)

        rounds(rot_a)
        x0 += k1
        x1 += np.uint32(k2 + np.uint32(1))
        rounds(rot_b)
        x0 += k2
        x1 += np.uint32(k0 + np.uint32(2))
        rounds(rot_a)
        x0 += k0
        x1 += np.uint32(k1 + np.uint32(3))
        rounds(rot_b)
        x0 += k1
        x1 += np.uint32(k2 + np.uint32(4))
        rounds(rot_a)
        x0 += k2
        x1 += np.uint32(k0 + np.uint32(5))
        x0 ^= x1
        x0 >>= np.uint32(9)
        np.less(x0, np.uint32(thresh_mant), out=out[s:e])
    return out


def _dropout_mask_words() -> np.ndarray:
    keep = _keep_mask_flat(0, 42, B * L * D, int(0.75 * (1 << 23)))
    k4 = keep.reshape(BT, 128, L, D).transpose(0, 2, 3, 1)
    packed = np.packbits(
        k4.reshape(BT, L, D, 4, 32).astype(np.uint8), axis=-1, bitorder="little"
    )
    return (
        packed.reshape(BT * L * D * 4, 4).view(np.uint32).astype(np.int32).reshape(-1)
    )


_MASK_WORDS = _dropout_mask_words()


@functools.partial(
    pl.kernel,
    out_type=jax.ShapeDtypeStruct((L, 8, BT, 1024), jnp.float32),
    mesh=plsc.VectorSubcoreMesh(core_axis_name="c", subcore_axis_name="s"),
    compiler_params=pltpu.CompilerParams(
        use_tc_tiling_on_sc=False, needs_layout_passes=False
    ),
    scratch_types=[
        pltpu.VMEM((1024,), jnp.int32),
        pltpu.VMEM((2048,), jnp.int32),
        pltpu.VMEM((8, 128, D), jnp.float32),
        pltpu.VMEM((128, 65), jnp.float32),
        pltpu.VMEM((2, 8, 1024), jnp.float32),
        pltpu.SemaphoreType.DMA((8,)),
        pltpu.SemaphoreType.DMA((2,)),
    ],
)
def _emb_dropout(
    xp_hbm, mw_hbm, table_hbm, out_hbm, idx_v, mw_v, rows_v, rp_v, out_v, sem_g, sem_o
):
    wid = lax.axis_index("s") * 2 + lax.axis_index("c")
    iota = lax.iota(jnp.int32, 16)
    row_idx = [iota + bg * 16 for bg in range(8)]
    shl = [31 - iota, 15 - iota]
    zero = jnp.zeros((16,), jnp.float32)
    scale = jnp.full((16,), SCALE, jnp.float32)

    def j_body(j, cj):
        bblk = wid * BPW + j

        def lt_body(lt, c0, j=j, bblk=bblk):
            pltpu.sync_copy(xp_hbm.at[pl.ds(((lt * BT + bblk) * 8) * 128, 1024)], idx_v)
            pltpu.sync_copy(mw_hbm.at[pl.ds((bblk * L + lt * 8) * 256, 2048)], mw_v)
            gathers = [
                pltpu.async_copy(
                    table_hbm.at[idx_v.at[pl.ds(k * 128, 128)]],
                    rows_v.at[k],
                    sem_g.at[k],
                )
                for k in range(8)
            ]
            for k in range(8):
                gathers[k].wait()

                def rp_body(rq, c3, k=k):
                    for rr in range(4):
                        r = rq * 4 + rr
                        for c in range(4):
                            rp_v[r, pl.ds(c * 16, 16)] = rows_v[k, r, pl.ds(c * 16, 16)]
                    return c3

                lax.fori_loop(0, 32, rp_body, 0)

                def dblk_body(db, c2, k=k):
                    ww = mw_v[pl.ds(k * 256 + db * 16, 16)]
                    for di in range(4):
                        d = db * 4 + di
                        col = jnp.broadcast_to(d, (16,))
                        for g in range(4):
                            w = jnp.broadcast_to(ww[di * 4 + g], (16,))
                            for half in range(2):
                                bg = g * 2 + half
                                v = plsc.load_gather(rp_v, [row_idx[bg], col])
                                keep = lax.shift_left(w, shl[half]) < 0
                                sv = lax.select(keep, v * scale, zero)
                                out_v[k % 2, d // 8, pl.ds((d % 8) * 128 + bg * 16, 16)] = sv
                    return c2

                l = lt * 8 + k
                if k >= 2:
                    pltpu.make_async_copy(
                        out_v.at[k % 2], out_hbm.at[l - 2, slice(None), bblk], sem_o.at[k % 2]
                    ).wait()
                else:

                    @pl.when(j + lt > 0)
                    def _(l=l, k=k, bblk=bblk):
                        pltpu.make_async_copy(
                            out_v.at[k % 2], out_hbm.at[l, slice(None), bblk], sem_o.at[k % 2]
                        ).wait()

                lax.fori_loop(0, 16, dblk_body, 0)
                pltpu.async_copy(
                    out_v.at[k % 2], out_hbm.at[l, slice(None), bblk], sem_o.at[k % 2]
                )
            return c0

        lax.fori_loop(0, LT, lt_body, 0)
        return cj

    lax.fori_loop(0, BPW, j_body, 0)

    for k in range(2):
        pltpu.make_async_copy(
            out_v.at[k], out_hbm.at[0, slice(None), 0], sem_o.at[k]
        ).wait()


def kernel(x, table):
    xp = (
        x.astype(jnp.int32)
        .transpose(1, 0)
        .reshape(LT, 8, BT, 128)
        .transpose(0, 2, 1, 3)
        .reshape(-1)
    )
    out5 = _emb_dropout(xp, _MASK_WORDS, table)
    return (
        out5.reshape(L, 8, BT, 8, 128).transpose(2, 4, 0, 1, 3).reshape(B, L, D)
    )


# speedup vs baseline: 2.6617x; 1.9725x over previous
import functools

import jax
import jax.numpy as jnp
import numpy as np
from jax import lax
from jax.experimental import pallas as pl
from jax.experimental.pallas import tpu as pltpu
from jax.experimental.pallas import tpu_sc as plsc

B, L, D = 16384, 200, 64
NROWS = B * L
NW = 32
LT = L // 8
BT = B // 128
BPW = BT // NW
SCALE = 1.0 / 0.75


def _keep_mask_flat(seed_hi: int, seed_lo: int, n: int, thresh_mant: int) -> np.ndarray:
    rot_a = (13, 15, 26, 6)
    rot_b = (17, 29, 16, 24)
    k0 = np.uint32(seed_hi)
    k1 = np.uint32(seed_lo)
    k2 = np.uint32(k0 ^ k1 ^ np.uint32(0x1BD11BDA))
    out = np.empty(n, dtype=bool)
    chunk = 1 << 24
    tmp = np.empty(chunk, dtype=np.uint32)
    for s in range(0, n, chunk):
        e = min(s + chunk, n)
        m = e - s
        x1 = np.arange(s, e, dtype=np.uint32)
        x0 = np.full(m, k0, dtype=np.uint32)
        x1 += k1
        t = tmp[:m]

        def rounds(rots):
            for r in rots:
                np.add(x0, x1, out=x0)
                np.left_shift(x1, np.uint32(r), out=t)
                np.right_shift(x1, np.uint32(32 - r), out=x1)
                np.bitwise_or(x1, t, out=x1)
                np.bitwise_xor(x1, x0, out=x1)

        rounds(rot_a)
        x0 += k1
        x1 += np.uint32(k2 + np.uint32(1))
        rounds(rot_b)
        x0 += k2
        x1 += np.uint32(k0 + np.uint32(2))
        rounds(rot_a)
        x0 += k0
        x1 += np.uint32(k1 + np.uint32(3))
        rounds(rot_b)
        x0 += k1
        x1 += np.uint32(k2 + np.uint32(4))
        rounds(rot_a)
        x0 += k2
        x1 += np.uint32(k0 + np.uint32(5))
        x0 ^= x1
        x0 >>= np.uint32(9)
        np.less(x0, np.uint32(thresh_mant), out=out[s:e])
    return out


def _dropout_mask_words() -> np.ndarray:
    keep = _keep_mask_flat(0, 42, B * L * D, int(0.75 * (1 << 23)))
    k4 = keep.reshape(BT, 128, L, D).transpose(0, 2, 3, 1)
    packed = np.packbits(
        k4.reshape(BT, L, D, 4, 32).astype(np.uint8), axis=-1, bitorder="little"
    )
    return (
        packed.reshape(BT * L * D * 4, 4).view(np.uint32).astype(np.int32).reshape(-1)
    )


_MASK_WORDS = _dropout_mask_words()


@functools.partial(
    pl.kernel,
    out_type=jax.ShapeDtypeStruct((L, 8, BT, 1024), jnp.float32),
    mesh=plsc.VectorSubcoreMesh(core_axis_name="c", subcore_axis_name="s"),
    compiler_params=pltpu.CompilerParams(
        use_tc_tiling_on_sc=False, needs_layout_passes=False
    ),
    scratch_types=[
        pltpu.VMEM((1024,), jnp.int32),
        pltpu.VMEM((2048,), jnp.int32),
        pltpu.VMEM((8, 128, D), jnp.float32),
        pltpu.VMEM((128, 65), jnp.float32),
        pltpu.VMEM((2, 8, 1024), jnp.float32),
        pltpu.SemaphoreType.DMA((8,)),
        pltpu.SemaphoreType.DMA((2,)),
    ],
)
def _emb_dropout(
    xp_hbm, mw_hbm, table_hbm, out_hbm, idx_v, mw_v, rows_v, rp_v, out_v, sem_g, sem_o
):
    wid = lax.axis_index("s") * 2 + lax.axis_index("c")
    iota = lax.iota(jnp.int32, 16)
    row_idx = [iota + bg * 16 for bg in range(8)]
    shl = [31 - iota, 15 - iota]
    zero = jnp.zeros((16,), jnp.float32)
    scale = jnp.full((16,), SCALE, jnp.float32)

    def j_body(j, cj):
        bblk = wid * BPW + j

        def lt_body(lt, c0, j=j, bblk=bblk):
            pltpu.sync_copy(xp_hbm.at[pl.ds(((lt * BT + bblk) * 8) * 128, 1024)], idx_v)
            pltpu.sync_copy(mw_hbm.at[pl.ds((bblk * L + lt * 8) * 256, 2048)], mw_v)
            gathers = [
                pltpu.async_copy(
                    table_hbm.at[idx_v.at[pl.ds(k * 128, 128)]],
                    rows_v.at[k],
                    sem_g.at[k],
                )
                for k in range(8)
            ]
            for k in range(8):
                gathers[k].wait()

                @plsc.parallel_loop(0, 32)
                def rp_body(rq, k=k):
                    for rr in range(4):
                        r = rq * 4 + rr
                        for c in range(4):
                            rp_v[r, pl.ds(c * 16, 16)] = rows_v[k, r, pl.ds(c * 16, 16)]

                def dblk_loop(k=k):
                    @plsc.parallel_loop(0, 16)
                    def dblk_body(db, k=k):
                        ww = mw_v[pl.ds(k * 256 + db * 16, 16)]
                        for di in range(4):
                            d = db * 4 + di
                            col = jnp.broadcast_to(d, (16,))
                            for g in range(4):
                                w = jnp.broadcast_to(ww[di * 4 + g], (16,))
                                for half in range(2):
                                    bg = g * 2 + half
                                    v = plsc.load_gather(rp_v, [row_idx[bg], col])
                                    keep = lax.shift_left(w, shl[half]) < 0
                                    sv = lax.select(keep, v * scale, zero)
                                    out_v[
                                        k % 2, d // 8, pl.ds((d % 8) * 128 + bg * 16, 16)
                                    ] = sv

                l = lt * 8 + k
                if k >= 2:
                    pltpu.make_async_copy(
                        out_v.at[k % 2], out_hbm.at[l - 2, slice(None), bblk], sem_o.at[k % 2]
                    ).wait()
                else:

                    @pl.when(j + lt > 0)
                    def _(l=l, k=k, bblk=bblk):
                        pltpu.make_async_copy(
                            out_v.at[k % 2], out_hbm.at[l, slice(None), bblk], sem_o.at[k % 2]
                        ).wait()

                dblk_loop()
                pltpu.async_copy(
                    out_v.at[k % 2], out_hbm.at[l, slice(None), bblk], sem_o.at[k % 2]
                )
            return c0

        lax.fori_loop(0, LT, lt_body, 0)
        return cj

    lax.fori_loop(0, BPW, j_body, 0)

    for k in range(2):
        pltpu.make_async_copy(
            out_v.at[k], out_hbm.at[0, slice(None), 0], sem_o.at[k]
        ).wait()


def kernel(x, table):
    xp = (
        x.astype(jnp.int32)
        .transpose(1, 0)
        .reshape(LT, 8, BT, 128)
        .transpose(0, 2, 1, 3)
        .reshape(-1)
    )
    out5 = _emb_dropout(xp, _MASK_WORDS, table)
    return (
        out5.reshape(L, 8, BT, 8, 128).transpose(2, 4, 0, 1, 3).reshape(B, L, D)
    )


# speedup vs baseline: 2.8748x; 1.0800x over previous
import functools

import jax
import jax.numpy as jnp
import numpy as np
from jax import lax
from jax.experimental import pallas as pl
from jax.experimental.pallas import tpu as pltpu
from jax.experimental.pallas import tpu_sc as plsc

B, L, D = 16384, 200, 64
NROWS = B * L
NW = 32
LT = L // 8
BT = B // 128
BPW = BT // NW
SCALE = 1.0 / 0.75


def _keep_mask_flat(seed_hi: int, seed_lo: int, n: int, thresh_mant: int) -> np.ndarray:
    rot_a = (13, 15, 26, 6)
    rot_b = (17, 29, 16, 24)
    k0 = np.uint32(seed_hi)
    k1 = np.uint32(seed_lo)
    k2 = np.uint32(k0 ^ k1 ^ np.uint32(0x1BD11BDA))
    out = np.empty(n, dtype=bool)
    chunk = 1 << 24
    tmp = np.empty(chunk, dtype=np.uint32)
    for s in range(0, n, chunk):
        e = min(s + chunk, n)
        m = e - s
        x1 = np.arange(s, e, dtype=np.uint32)
        x0 = np.full(m, k0, dtype=np.uint32)
        x1 += k1
        t = tmp[:m]

        def rounds(rots):
            for r in rots:
                np.add(x0, x1, out=x0)
                np.left_shift(x1, np.uint32(r), out=t)
                np.right_shift(x1, np.uint32(32 - r), out=x1)
                np.bitwise_or(x1, t, out=x1)
                np.bitwise_xor(x1, x0, out=x1)

        rounds(rot_a)
        x0 += k1
        x1 += np.uint32(k2 + np.uint32(1))
        rounds(rot_b)
        x0 += k2
        x1 += np.uint32(k0 + np.uint32(2))
        rounds(rot_a)
        x0 += k0
        x1 += np.uint32(k1 + np.uint32(3))
        rounds(rot_b)
        x0 += k1
        x1 += np.uint32(k2 + np.uint32(4))
        rounds(rot_a)
        x0 += k2
        x1 += np.uint32(k0 + np.uint32(5))
        x0 ^= x1
        x0 >>= np.uint32(9)
        np.less(x0, np.uint32(thresh_mant), out=out[s:e])
    return out


def _dropout_mask_words() -> np.ndarray:
    keep = _keep_mask_flat(0, 42, B * L * D, int(0.75 * (1 << 23)))
    k4 = keep.reshape(BT, 128, L, D).transpose(0, 2, 3, 1)
    packed = np.packbits(
        k4.reshape(BT, L, D, 4, 32).astype(np.uint8), axis=-1, bitorder="little"
    )
    return (
        packed.reshape(BT * L * D * 4, 4).view(np.uint32).astype(np.int32).reshape(-1)
    )


_MASK_WORDS = _dropout_mask_words()


@functools.partial(
    pl.kernel,
    out_type=jax.ShapeDtypeStruct((L, 8, BT, 1024), jnp.float32),
    mesh=plsc.VectorSubcoreMesh(core_axis_name="c", subcore_axis_name="s"),
    compiler_params=pltpu.CompilerParams(
        use_tc_tiling_on_sc=False, needs_layout_passes=False
    ),
    scratch_types=[
        pltpu.VMEM((2, 1024), jnp.int32),
        pltpu.VMEM((2, 2048), jnp.int32),
        pltpu.VMEM((8, 128, D), jnp.float32),
        pltpu.VMEM((128, 65), jnp.float32),
        pltpu.VMEM((2, 8, 1024), jnp.float32),
        pltpu.SemaphoreType.DMA((8,)),
        pltpu.SemaphoreType.DMA((2,)),
        pltpu.SemaphoreType.DMA((2,)),
        pltpu.SemaphoreType.DMA((2,)),
    ],
)
def _emb_dropout(
    xp_hbm,
    mw_hbm,
    table_hbm,
    out_hbm,
    idx_v,
    mw_v,
    rows_v,
    rp_v,
    out_v,
    sem_g,
    sem_o,
    sem_i,
    sem_m,
):
    wid = lax.axis_index("s") * 2 + lax.axis_index("c")
    iota = lax.iota(jnp.int32, 16)
    row_idx = [iota + bg * 16 for bg in range(8)]
    shl = [31 - iota, 15 - iota]
    zero = jnp.zeros((16,), jnp.float32)
    scale = jnp.full((16,), SCALE, jnp.float32)

    bblk0 = wid * BPW
    pltpu.async_copy(xp_hbm.at[pl.ds(bblk0 * 1024, 1024)], idx_v.at[0], sem_i.at[0])
    pltpu.async_copy(mw_hbm.at[pl.ds(bblk0 * L * 256, 2048)], mw_v.at[0], sem_m.at[0])

    def j_body(j, cj):
        bblk = wid * BPW + j

        def lt_body(lt, c0, j=j, bblk=bblk):
            t = j * LT + lt
            par = t % 2
            pltpu.make_async_copy(
                xp_hbm.at[pl.ds(0, 1024)], idx_v.at[par], sem_i.at[par]
            ).wait()
            gathers = [
                pltpu.async_copy(
                    table_hbm.at[idx_v.at[par, pl.ds(k * 128, 128)]],
                    rows_v.at[k],
                    sem_g.at[k],
                )
                for k in range(8)
            ]

            @pl.when(t + 1 < BPW * LT)
            def _(t=t, par=par):
                t2 = t + 1
                bblk2 = wid * BPW + t2 // LT
                lt2 = t2 % LT
                pltpu.async_copy(
                    xp_hbm.at[pl.ds(((lt2 * BT + bblk2) * 8) * 128, 1024)],
                    idx_v.at[1 - par],
                    sem_i.at[1 - par],
                )
                pltpu.async_copy(
                    mw_hbm.at[pl.ds((bblk2 * L + lt2 * 8) * 256, 2048)],
                    mw_v.at[1 - par],
                    sem_m.at[1 - par],
                )

            pltpu.make_async_copy(
                mw_hbm.at[pl.ds(0, 2048)], mw_v.at[par], sem_m.at[par]
            ).wait()
            for k in range(8):
                gathers[k].wait()

                @plsc.parallel_loop(0, 32)
                def rp_body(rq, k=k):
                    for rr in range(4):
                        r = rq * 4 + rr
                        for c in range(4):
                            rp_v[r, pl.ds(c * 16, 16)] = rows_v[k, r, pl.ds(c * 16, 16)]

                def dblk_loop(k=k, par=par):
                    @plsc.parallel_loop(0, 16)
                    def dblk_body(db, k=k, par=par):
                        ww = mw_v[par, pl.ds(k * 256 + db * 16, 16)]
                        for di in range(4):
                            d = db * 4 + di
                            col = jnp.broadcast_to(d, (16,))
                            for g in range(4):
                                w = jnp.broadcast_to(ww[di * 4 + g], (16,))
                                for half in range(2):
                                    bg = g * 2 + half
                                    v = plsc.load_gather(rp_v, [row_idx[bg], col])
                                    keep = lax.shift_left(w, shl[half]) < 0
                                    sv = lax.select(keep, v * scale, zero)
                                    out_v[
                                        k % 2, d // 8, pl.ds((d % 8) * 128 + bg * 16, 16)
                                    ] = sv

                l = lt * 8 + k
                if k >= 2:
                    pltpu.make_async_copy(
                        out_v.at[k % 2], out_hbm.at[l - 2, slice(None), bblk], sem_o.at[k % 2]
                    ).wait()
                else:

                    @pl.when(j + lt > 0)
                    def _(l=l, k=k, bblk=bblk):
                        pltpu.make_async_copy(
                            out_v.at[k % 2], out_hbm.at[l, slice(None), bblk], sem_o.at[k % 2]
                        ).wait()

                dblk_loop()
                pltpu.async_copy(
                    out_v.at[k % 2], out_hbm.at[l, slice(None), bblk], sem_o.at[k % 2]
                )
            return c0

        lax.fori_loop(0, LT, lt_body, 0)
        return cj

    lax.fori_loop(0, BPW, j_body, 0)

    for k in range(2):
        pltpu.make_async_copy(
            out_v.at[k], out_hbm.at[0, slice(None), 0], sem_o.at[k]
        ).wait()


def kernel(x, table):
    xp = (
        x.astype(jnp.int32)
        .transpose(1, 0)
        .reshape(LT, 8, BT, 128)
        .transpose(0, 2, 1, 3)
        .reshape(-1)
    )
    out5 = _emb_dropout(xp, _MASK_WORDS, table)
    return (
        out5.reshape(L, 8, BT, 8, 128).transpose(2, 4, 0, 1, 3).reshape(B, L, D)
    )


# speedup vs baseline: 3.3080x; 1.1507x over previous
import functools

import jax
import jax.numpy as jnp
import numpy as np
from jax import lax
from jax.experimental import pallas as pl
from jax.experimental.pallas import tpu as pltpu
from jax.experimental.pallas import tpu_sc as plsc

B, L, D = 16384, 200, 64
NROWS = B * L
NW = 32
LT = L // 8
BT = B // 128
BPW = BT // NW
SCALE = 1.0 / 0.75


def _keep_mask_flat(seed_hi: int, seed_lo: int, n: int, thresh_mant: int) -> np.ndarray:
    rot_a = (13, 15, 26, 6)
    rot_b = (17, 29, 16, 24)
    k0 = np.uint32(seed_hi)
    k1 = np.uint32(seed_lo)
    k2 = np.uint32(k0 ^ k1 ^ np.uint32(0x1BD11BDA))
    out = np.empty(n, dtype=bool)
    chunk = 1 << 24
    tmp = np.empty(chunk, dtype=np.uint32)
    for s in range(0, n, chunk):
        e = min(s + chunk, n)
        m = e - s
        x1 = np.arange(s, e, dtype=np.uint32)
        x0 = np.full(m, k0, dtype=np.uint32)
        x1 += k1
        t = tmp[:m]

        def rounds(rots):
            for r in rots:
                np.add(x0, x1, out=x0)
                np.left_shift(x1, np.uint32(r), out=t)
                np.right_shift(x1, np.uint32(32 - r), out=x1)
                np.bitwise_or(x1, t, out=x1)
                np.bitwise_xor(x1, x0, out=x1)

        rounds(rot_a)
        x0 += k1
        x1 += np.uint32(k2 + np.uint32(1))
        rounds(rot_b)
        x0 += k2
        x1 += np.uint32(k0 + np.uint32(2))
        rounds(rot_a)
        x0 += k0
        x1 += np.uint32(k1 + np.uint32(3))
        rounds(rot_b)
        x0 += k1
        x1 += np.uint32(k2 + np.uint32(4))
        rounds(rot_a)
        x0 += k2
        x1 += np.uint32(k0 + np.uint32(5))
        x0 ^= x1
        x0 >>= np.uint32(9)
        np.less(x0, np.uint32(thresh_mant), out=out[s:e])
    return out


def _dropout_mask_words() -> np.ndarray:
    keep = _keep_mask_flat(0, 42, B * L * D, int(0.75 * (1 << 23)))
    k4 = keep.reshape(BT, 128, L, D).transpose(0, 2, 1, 3)
    packed = np.packbits(
        k4.reshape(BT, L, 128, 2, 32).astype(np.uint8), axis=-1, bitorder="little"
    )
    return (
        packed.reshape(BT * L * 128 * 2, 4).view(np.uint32).astype(np.int32).reshape(-1)
    )


_MASK_WORDS = _dropout_mask_words()


@functools.partial(
    pl.kernel,
    out_type=jax.ShapeDtypeStruct((L, 8, BT, 8, 128), jnp.float32),
    mesh=plsc.VectorSubcoreMesh(core_axis_name="c", subcore_axis_name="s"),
    compiler_params=pltpu.CompilerParams(
        use_tc_tiling_on_sc=False, needs_layout_passes=False
    ),
    scratch_types=[
        pltpu.VMEM((2, 1024), jnp.int32),
        pltpu.VMEM((2, 2048), jnp.int32),
        pltpu.VMEM((8, 128, D), jnp.float32),
        pltpu.VMEM((8, 8, 129), jnp.float32),
        pltpu.VMEM((8, 8, 129), jnp.float32),
        pltpu.SemaphoreType.DMA((8,)),
        pltpu.SemaphoreType.DMA((2,)),
        pltpu.SemaphoreType.DMA((2,)),
        pltpu.SemaphoreType.DMA((2,)),
    ],
)
def _emb_dropout(
    xp_hbm,
    mw_hbm,
    table_hbm,
    out_hbm,
    idx_v,
    mw_v,
    rows_v,
    out_va,
    out_vb,
    sem_g,
    sem_o,
    sem_i,
    sem_m,
):
    wid = lax.axis_index("s") * 2 + lax.axis_index("c")
    iota = lax.iota(jnp.int32, 16)
    tdc = [(iota + 16 * c) // 8 for c in range(4)]
    dsc = [(iota + 16 * c) % 8 for c in range(4)]
    shl = [31 - iota, 15 - iota]
    zero = jnp.zeros((16,), jnp.float32)
    scale = jnp.full((16,), SCALE, jnp.float32)
    out_bufs = [out_va, out_vb]

    bblk0 = wid * BPW
    pltpu.async_copy(xp_hbm.at[pl.ds(bblk0 * 1024, 1024)], idx_v.at[0], sem_i.at[0])
    pltpu.async_copy(mw_hbm.at[pl.ds(bblk0 * L * 256, 2048)], mw_v.at[0], sem_m.at[0])

    def j_body(j, cj):
        bblk = wid * BPW + j

        def lt_body(lt, c0, j=j, bblk=bblk):
            t = j * LT + lt
            par = t % 2
            pltpu.make_async_copy(
                xp_hbm.at[pl.ds(0, 1024)], idx_v.at[par], sem_i.at[par]
            ).wait()
            gathers = [
                pltpu.async_copy(
                    table_hbm.at[idx_v.at[par, pl.ds(k * 128, 128)]],
                    rows_v.at[k],
                    sem_g.at[k],
                )
                for k in range(8)
            ]

            @pl.when(t + 1 < BPW * LT)
            def _(t=t, par=par):
                t2 = t + 1
                bblk2 = wid * BPW + t2 // LT
                lt2 = t2 % LT
                pltpu.async_copy(
                    xp_hbm.at[pl.ds(((lt2 * BT + bblk2) * 8) * 128, 1024)],
                    idx_v.at[1 - par],
                    sem_i.at[1 - par],
                )
                pltpu.async_copy(
                    mw_hbm.at[pl.ds((bblk2 * L + lt2 * 8) * 256, 2048)],
                    mw_v.at[1 - par],
                    sem_m.at[1 - par],
                )

            pltpu.make_async_copy(
                mw_hbm.at[pl.ds(0, 2048)], mw_v.at[par], sem_m.at[par]
            ).wait()
            for k in range(8):
                gathers[k].wait()
                ob = out_bufs[k % 2]
                obsrc = ob.at[slice(None), slice(None), pl.ds(0, 128)]
                l = lt * 8 + k
                if k >= 2:
                    pltpu.make_async_copy(
                        obsrc, out_hbm.at[l - 2, slice(None), bblk], sem_o.at[k % 2]
                    ).wait()
                else:

                    @pl.when(j + lt > 0)
                    def _(l=l, obsrc=obsrc, k=k, bblk=bblk):
                        pltpu.make_async_copy(
                            obsrc, out_hbm.at[l, slice(None), bblk], sem_o.at[k % 2]
                        ).wait()

                @plsc.parallel_loop(0, 16)
                def row_body(rb, k=k, par=par, ob=ob):
                    ww = mw_v[par, pl.ds(k * 256 + rb * 16, 16)]
                    for rr in range(8):
                        b = rb * 8 + rr
                        bs = jnp.broadcast_to(b, (16,))
                        w0 = jnp.broadcast_to(ww[2 * rr], (16,))
                        w1 = jnp.broadcast_to(ww[2 * rr + 1], (16,))
                        for c in range(4):
                            v = rows_v[k, b, pl.ds(c * 16, 16)]
                            w = w0 if c < 2 else w1
                            keep = lax.shift_left(w, shl[c % 2]) < 0
                            sv = lax.select(keep, v * scale, zero)
                            plsc.store_scatter(ob, [tdc[c], dsc[c], bs], sv)

                pltpu.async_copy(
                    obsrc, out_hbm.at[l, slice(None), bblk], sem_o.at[k % 2]
                )
            return c0

        lax.fori_loop(0, LT, lt_body, 0)
        return cj

    lax.fori_loop(0, BPW, j_body, 0)

    for k in range(2):
        pltpu.make_async_copy(
            out_bufs[k].at[slice(None), slice(None), pl.ds(0, 128)],
            out_hbm.at[0, slice(None), 0],
            sem_o.at[k],
        ).wait()


def kernel(x, table):
    xp = (
        x.astype(jnp.int32)
        .transpose(1, 0)
        .reshape(LT, 8, BT, 128)
        .transpose(0, 2, 1, 3)
        .reshape(-1)
    )
    out5 = _emb_dropout(xp, _MASK_WORDS, table)
    return (
        out5.transpose(2, 4, 0, 1, 3).reshape(B, L, D)
    )
